# Initial kernel scaffold; baseline (speedup 1.0000x reference)
import jax
import jax.numpy as jnp
from jax.experimental import pallas as pl


def kernel(x, binpoints, w_ih_0, w_hh_0, b_ih_0, b_hh_0, w_ih_1, w_hh_1, b_ih_1, b_hh_1, w_ih_2, w_hh_2, b_ih_2, b_hh_2):
    raise NotImplementedError("write your pallas kernel here")


# speedup vs baseline: 8.7801x; 8.7801x over previous
import functools

import jax
import jax.numpy as jnp
from jax.experimental import pallas as pl
from jax.experimental.pallas import tpu as pltpu

_NFILT = 40
_FPAD = 64


def _frontend_body(nfilt, b0_ref, b1_ref, b2_ref, x_ref, o_ref):
    nb = x_ref.shape[-1]
    fp = o_ref.shape[-1]
    b0, b1, b2 = b0_ref[...], b1_ref[...], b2_ref[...]
    f0, f1, f2 = jnp.floor(b0), jnp.floor(b1), jnp.floor(b2)
    i = jax.lax.broadcasted_iota(jnp.int32, (nb, fp), 0).astype(jnp.float32)
    j = jax.lax.broadcasted_iota(jnp.int32, (nb, fp), 1)
    rise_m = (i >= f0) & (i < f1)
    fall_m = (i >= f1) & (i < f2)
    d1 = b1 - b0
    d2 = b2 - b1
    rv = (i - b0) / jnp.where(d1 > 0, d1, 1.0) ** 2
    fv = (b2 - i) / jnp.where(d2 > 0, d2, 1.0) ** 2
    val = jnp.where(fall_m, fv, jnp.where(rise_m, rv, 0.0))
    val = jnp.where(j < nfilt - 1, val, 0.0)
    val = jnp.where(j == 0, jnp.where(i == 0.0, 1.0, 0.0), val)

    bh, tc, _ = x_ref.shape
    xb = x_ref[...].reshape(bh * tc, nb)
    filt = jnp.dot(xb, val, preferred_element_type=jnp.float32)
    o_ref[...] = jnp.log(filt + 1e-10).reshape(bh, tc, fp)


def _frontend(x, binpoints, bh, tcf):
    B, T, NB = x.shape
    nt = T // tcf
    pad = _FPAD - _NFILT
    b0 = jnp.pad(binpoints[0:_NFILT], (0, pad)).reshape(1, _FPAD)
    b1 = jnp.pad(binpoints[1:_NFILT + 1], (0, pad)).reshape(1, _FPAD)
    b2 = jnp.pad(binpoints[2:_NFILT + 2], (0, pad)).reshape(1, _FPAD)
    return pl.pallas_call(
        functools.partial(_frontend_body, _NFILT),
        grid=(B // bh, nt),
        in_specs=[
            pl.BlockSpec((1, _FPAD), lambda b, k: (0, 0)),
            pl.BlockSpec((1, _FPAD), lambda b, k: (0, 0)),
            pl.BlockSpec((1, _FPAD), lambda b, k: (0, 0)),
            pl.BlockSpec((bh, tcf, NB), lambda b, k: (b, k, 0)),
        ],
        out_specs=pl.BlockSpec((bh, tcf, _FPAD), lambda b, k: (b, k, 0)),
        out_shape=jax.ShapeDtypeStruct((B, T, _FPAD), jnp.float32),
        compiler_params=pltpu.CompilerParams(
            dimension_semantics=("parallel", "arbitrary")),
    )(b0, b1, b2, x)


def _lstm_body(tc, bh, hid, n_in, accumulate, t_total, nt, *refs):
    xf = refs[0:n_in]
    xb = refs[n_in:2 * n_in]
    wif, wib, whf, whb, bf, bb = refs[2 * n_in:2 * n_in + 6]
    of, ob = refs[2 * n_in + 6], refs[2 * n_in + 7]
    pf_s, pb_s, st_s = refs[2 * n_in + 8:2 * n_in + 11]
    k = pl.program_id(1)

    @pl.when(k == 0)
    def _():
        st_s[...] = jnp.zeros_like(st_s)

    def make_pre(xs, w_ref, b_ref):
        acc = None
        for i, xr in enumerate(xs):
            d = xr.shape[-1]
            x2 = xr[...].reshape(tc * bh, d)
            p = jnp.dot(x2, w_ref[i * d:(i + 1) * d, :],
                        preferred_element_type=jnp.float32)
            acc = p if acc is None else acc + p
        return acc + b_ref[...]

    pf_s[...] = make_pre(xf, wif, bf)
    pb_s[...] = make_pre(xb, wib, bb)

    whf_v = whf[...]
    whb_v = whb[...]
    st = st_s[...]

    def gates(g, c):
        ig = jax.nn.sigmoid(g[:, 0:hid])
        fg = jax.nn.sigmoid(g[:, hid:2 * hid])
        gg = jnp.tanh(g[:, 2 * hid:3 * hid])
        og = jax.nn.sigmoid(g[:, 3 * hid:4 * hid])
        c = fg * c + ig * gg
        h = og * jnp.tanh(c)
        return h, c

    def step(t, carry):
        h_f, c_f, h_b, c_b, a_f, a_b = carry
        rf = pl.multiple_of(t * bh, bh)
        gf = pf_s[pl.ds(rf, bh), :] + jnp.dot(
            h_f, whf_v, preferred_element_type=jnp.float32)
        h_f, c_f = gates(gf, c_f)
        tb = tc - 1 - t
        rb = pl.multiple_of(tb * bh, bh)
        gb = pb_s[pl.ds(rb, bh), :] + jnp.dot(
            h_b, whb_v, preferred_element_type=jnp.float32)
        h_b, c_b = gates(gb, c_b)
        if accumulate:
            a_f = a_f + h_f
            a_b = a_b + h_b
        else:
            of[pl.ds(t, 1)] = h_f.reshape(1, bh, hid)
            ob[pl.ds(tb, 1)] = h_b.reshape(1, bh, hid)
        return (h_f, c_f, h_b, c_b, a_f, a_b)

    init = tuple(st[i] for i in range(6))
    fin = jax.lax.fori_loop(0, tc, step, init)
    for i in range(6):
        st_s[i] = fin[i]

    if accumulate:
        @pl.when(k == nt - 1)
        def _():
            of[...] = fin[4] * (1.0 / t_total)
            ob[...] = fin[5] * (1.0 / t_total)


def _lstm_layer(ins, wiT, whT, bsum, bh, tc, accumulate):
    T, B, d = ins[0].shape
    hid = whT.shape[1]
    g4 = whT.shape[2]
    nt = T // tc
    nb = B // bh
    n_in = len(ins)

    in_specs = []
    args = []
    for xr in ins:
        in_specs.append(pl.BlockSpec((tc, bh, d), lambda b, k: (k, b, 0)))
        args.append(xr)
    for xr in ins:
        in_specs.append(
            pl.BlockSpec((tc, bh, d), lambda b, k: (nt - 1 - k, b, 0)))
        args.append(xr)
    for w in (wiT[0], wiT[1], whT[0], whT[1],
              bsum[0].reshape(1, g4), bsum[1].reshape(1, g4)):
        in_specs.append(pl.BlockSpec(w.shape, lambda b, k: (0,) * w.ndim))
        args.append(w)

    if accumulate:
        out_shape = (jax.ShapeDtypeStruct((B, hid), jnp.float32),) * 2
        out_specs = [pl.BlockSpec((bh, hid), lambda b, k: (b, 0)),
                     pl.BlockSpec((bh, hid), lambda b, k: (b, 0))]
    else:
        out_shape = (jax.ShapeDtypeStruct((T, B, hid), jnp.float32),) * 2
        out_specs = [
            pl.BlockSpec((tc, bh, hid), lambda b, k: (k, b, 0)),
            pl.BlockSpec((tc, bh, hid), lambda b, k: (nt - 1 - k, b, 0)),
        ]

    scratch = [
        pltpu.VMEM((tc * bh, g4), jnp.float32),
        pltpu.VMEM((tc * bh, g4), jnp.float32),
        pltpu.VMEM((6, bh, hid), jnp.float32),
    ]
    return pl.pallas_call(
        functools.partial(_lstm_body, tc, bh, hid, n_in, accumulate, T, nt),
        grid=(nb, nt),
        in_specs=in_specs,
        out_specs=out_specs,
        out_shape=out_shape,
        scratch_shapes=scratch,
        compiler_params=pltpu.CompilerParams(
            dimension_semantics=("parallel", "arbitrary"),
            vmem_limit_bytes=56 * 1024 * 1024),
    )(*args)


def kernel(x, binpoints, w_ih_0, w_hh_0, b_ih_0, b_hh_0,
           w_ih_1, w_hh_1, b_ih_1, b_hh_1,
           w_ih_2, w_hh_2, b_ih_2, b_hh_2):
    B, T, NB = x.shape
    hid = w_hh_0.shape[-1]
    bh = B // 2
    tcf = 200 if T % 200 == 0 else T
    tc = 100 if T % 100 == 0 else T

    h0 = _frontend(x, binpoints, bh, tcf)
    h0t = jnp.transpose(h0, (1, 0, 2))

    wi0T = jnp.pad(jnp.transpose(w_ih_0, (0, 2, 1)),
                   ((0, 0), (0, _FPAD - _NFILT), (0, 0)))
    wi1T = jnp.transpose(w_ih_1, (0, 2, 1))
    wi2T = jnp.transpose(w_ih_2, (0, 2, 1))
    wh0T = jnp.transpose(w_hh_0, (0, 2, 1))
    wh1T = jnp.transpose(w_hh_1, (0, 2, 1))
    wh2T = jnp.transpose(w_hh_2, (0, 2, 1))

    f0, r0 = _lstm_layer([h0t], wi0T, wh0T, b_ih_0 + b_hh_0, bh, tc, False)
    f1, r1 = _lstm_layer([f0, r0], wi1T, wh1T, b_ih_1 + b_hh_1, bh, tc, False)
    mf, mb = _lstm_layer([f1, r1], wi2T, wh2T, b_ih_2 + b_hh_2, bh, tc, True)
    return jnp.concatenate([mf, mb], axis=-1)


# speedup vs baseline: 13.4945x; 1.5369x over previous
import functools

import jax
import jax.numpy as jnp
from jax.experimental import pallas as pl
from jax.experimental.pallas import tpu as pltpu

_NFILT = 40
_FPAD = 64


def _frontend_body(nfilt, b0_ref, b1_ref, b2_ref, x_ref, o_ref):
    nb = x_ref.shape[-1]
    fp = o_ref.shape[-1]
    b0, b1, b2 = b0_ref[...], b1_ref[...], b2_ref[...]
    f0, f1, f2 = jnp.floor(b0), jnp.floor(b1), jnp.floor(b2)
    i = jax.lax.broadcasted_iota(jnp.int32, (nb, fp), 0).astype(jnp.float32)
    j = jax.lax.broadcasted_iota(jnp.int32, (nb, fp), 1)
    rise_m = (i >= f0) & (i < f1)
    fall_m = (i >= f1) & (i < f2)
    d1 = b1 - b0
    d2 = b2 - b1
    rv = (i - b0) / jnp.where(d1 > 0, d1, 1.0) ** 2
    fv = (b2 - i) / jnp.where(d2 > 0, d2, 1.0) ** 2
    val = jnp.where(fall_m, fv, jnp.where(rise_m, rv, 0.0))
    val = jnp.where(j < nfilt - 1, val, 0.0)
    val = jnp.where(j == 0, jnp.where(i == 0.0, 1.0, 0.0), val)

    bh, tc, _ = x_ref.shape
    xb = x_ref[...].reshape(bh * tc, nb)
    filt = jnp.dot(xb, val, preferred_element_type=jnp.float32)
    o_ref[...] = jnp.log(filt + 1e-10).reshape(bh, tc, fp)


def _frontend(x, binpoints, bh, tcf):
    B, T, NB = x.shape
    nt = T // tcf
    pad = _FPAD - _NFILT
    b0 = jnp.pad(binpoints[0:_NFILT], (0, pad)).reshape(1, _FPAD)
    b1 = jnp.pad(binpoints[1:_NFILT + 1], (0, pad)).reshape(1, _FPAD)
    b2 = jnp.pad(binpoints[2:_NFILT + 2], (0, pad)).reshape(1, _FPAD)
    return pl.pallas_call(
        functools.partial(_frontend_body, _NFILT),
        grid=(B // bh, nt),
        in_specs=[
            pl.BlockSpec((1, _FPAD), lambda b, k: (0, 0)),
            pl.BlockSpec((1, _FPAD), lambda b, k: (0, 0)),
            pl.BlockSpec((1, _FPAD), lambda b, k: (0, 0)),
            pl.BlockSpec((bh, tcf, NB), lambda b, k: (b, k, 0)),
        ],
        out_specs=pl.BlockSpec((bh, tcf, _FPAD), lambda b, k: (b, k, 0)),
        out_shape=jax.ShapeDtypeStruct((B, T, _FPAD), jnp.float32),
        compiler_params=pltpu.CompilerParams(
            dimension_semantics=("parallel", "arbitrary")),
    )(b0, b1, b2, x)


def _lstm_body(tc, bh, hid2, n_in, accumulate, t_total, nt, *refs):
    g8 = 4 * hid2
    xf = refs[0:n_in]
    xb = refs[n_in:2 * n_in]
    wf = refs[2 * n_in:3 * n_in]
    wb = refs[3 * n_in:4 * n_in]
    wc, bf, bb = refs[4 * n_in:4 * n_in + 3]
    n_out = 1 if accumulate else 2
    outs = refs[4 * n_in + 3:4 * n_in + 3 + n_out]
    pf_s, pb_s, st_s = refs[4 * n_in + 3 + n_out:]
    k = pl.program_id(1)

    @pl.when(k == 0)
    def _():
        st_s[...] = jnp.zeros_like(st_s)

    def make_pre(xs, ws, b_ref):
        acc = b_ref[...]
        for xr, w_ref in zip(xs, ws):
            d = xr.shape[-1]
            x2 = xr[...].reshape(tc * bh, d)
            acc = acc + jnp.dot(x2, w_ref[...],
                                preferred_element_type=jnp.float32)
        return acc

    pf_s[...] = make_pre(xf, wf, bf)
    pb_s[...] = make_pre(xb, wb, bb)

    wc_v = wc[...]
    st = st_s[...]

    def step(t, carry):
        h, c, a = carry
        rf = pl.multiple_of(t * bh, bh)
        tb = tc - 1 - t
        rb = pl.multiple_of(tb * bh, bh)
        g = (pf_s[pl.ds(rf, bh), :] + pb_s[pl.ds(rb, bh), :]
             + jnp.dot(h, wc_v, preferred_element_type=jnp.float32))
        ig = jax.nn.sigmoid(g[:, 0:hid2])
        fg = jax.nn.sigmoid(g[:, hid2:2 * hid2])
        gg = jnp.tanh(g[:, 2 * hid2:3 * hid2])
        og = jax.nn.sigmoid(g[:, 3 * hid2:4 * hid2])
        c = fg * c + ig * gg
        h = og * jnp.tanh(c)
        if accumulate:
            a = a + h
        else:
            outs[0][pl.ds(t, 1)] = h.reshape(1, bh, hid2)
            outs[1][pl.ds(tb, 1)] = h.reshape(1, bh, hid2)
        return (h, c, a)

    init = (st[0], st[1], st[2])
    fin = jax.lax.fori_loop(0, tc, step, init)
    for i in range(3):
        st_s[i] = fin[i]

    if accumulate:
        @pl.when(k == nt - 1)
        def _():
            outs[0][...] = fin[2] * (1.0 / t_total)


def _lstm_layer(ins, wf_list, wb_list, wc, bf, bb, bh, tc, accumulate):
    T, B, _ = ins[0].shape
    hid2 = wc.shape[0]
    g8 = wc.shape[1]
    nt = T // tc
    nb = B // bh
    n_in = len(ins)

    in_specs = []
    args = []
    for xr in ins:
        d = xr.shape[-1]
        in_specs.append(pl.BlockSpec((tc, bh, d), lambda b, k: (k, b, 0)))
        args.append(xr)
    for xr in ins:
        d = xr.shape[-1]
        in_specs.append(
            pl.BlockSpec((tc, bh, d), lambda b, k: (nt - 1 - k, b, 0)))
        args.append(xr)
    for w in (*wf_list, *wb_list, wc, bf, bb):
        in_specs.append(pl.BlockSpec(w.shape, lambda b, k: (0,) * w.ndim))
        args.append(w)

    if accumulate:
        out_shape = (jax.ShapeDtypeStruct((B, hid2), jnp.float32),)
        out_specs = [pl.BlockSpec((bh, hid2), lambda b, k: (b, 0))]
    else:
        out_shape = (jax.ShapeDtypeStruct((T, B, hid2), jnp.float32),) * 2
        out_specs = [
            pl.BlockSpec((tc, bh, hid2), lambda b, k: (k, b, 0)),
            pl.BlockSpec((tc, bh, hid2), lambda b, k: (nt - 1 - k, b, 0)),
        ]

    scratch = [
        pltpu.VMEM((tc * bh, g8), jnp.float32),
        pltpu.VMEM((tc * bh, g8), jnp.float32),
        pltpu.VMEM((3, bh, hid2), jnp.float32),
    ]
    out = pl.pallas_call(
        functools.partial(_lstm_body, tc, bh, hid2, n_in, accumulate, T, nt),
        grid=(nb, nt),
        in_specs=in_specs,
        out_specs=out_specs,
        out_shape=out_shape,
        scratch_shapes=scratch,
        compiler_params=pltpu.CompilerParams(
            dimension_semantics=("parallel", "arbitrary"),
            vmem_limit_bytes=56 * 1024 * 1024),
    )(*args)
    return out


def _spread(w, slot, hid):
    z = jnp.zeros(w.shape[:-1] + (hid,), w.dtype)
    parts = []
    for q in range(4):
        blk = w[..., q * hid:(q + 1) * hid]
        parts.extend([blk, z] if slot == 0 else [z, blk])
    return jnp.concatenate(parts, axis=-1)


def kernel(x, binpoints, w_ih_0, w_hh_0, b_ih_0, b_hh_0,
           w_ih_1, w_hh_1, b_ih_1, b_hh_1,
           w_ih_2, w_hh_2, b_ih_2, b_hh_2):
    B, T, NB = x.shape
    hid = w_hh_0.shape[-1]
    bh = B // 2
    tcf = 200 if T % 200 == 0 else T
    tc = 100 if T % 100 == 0 else T

    h0 = _frontend(x, binpoints, bh, tcf)
    h0t = jnp.transpose(h0, (1, 0, 2))

    def wiT(w):
        return jnp.transpose(w, (0, 2, 1))

    wi0T = jnp.pad(wiT(w_ih_0), ((0, 0), (0, _FPAD - _NFILT), (0, 0)))
    wi1T, wi2T = wiT(w_ih_1), wiT(w_ih_2)
    wh0T, wh1T, wh2T = wiT(w_hh_0), wiT(w_hh_1), wiT(w_hh_2)

    def combine_wh(whT):
        return jnp.concatenate(
            [_spread(whT[0], 0, hid), _spread(whT[1], 1, hid)], axis=0)

    def biases(b_ih, b_hh):
        bs = b_ih + b_hh
        return (_spread(bs[0].reshape(1, -1), 0, hid),
                _spread(bs[1].reshape(1, -1), 1, hid))

    def zero_rows(w, keep_top):
        top, bot = w[:hid], w[hid:]
        if keep_top:
            return jnp.concatenate([top, jnp.zeros_like(bot)], axis=0)
        return jnp.concatenate([jnp.zeros_like(top), bot], axis=0)

    bf0, bb0 = biases(b_ih_0, b_hh_0)
    f0, r0 = _lstm_layer(
        [h0t],
        [_spread(wi0T[0], 0, hid)], [_spread(wi0T[1], 1, hid)],
        combine_wh(wh0T), bf0, bb0, bh, tc, False)

    def mk_io_weights(wT):
        wfs = [_spread(zero_rows(wT[0], True), 0, hid),
               _spread(zero_rows(wT[0], False), 0, hid)]
        wbs = [_spread(zero_rows(wT[1], True), 1, hid),
               _spread(zero_rows(wT[1], False), 1, hid)]
        return wfs, wbs

    wfs1, wbs1 = mk_io_weights(wi1T)
    bf1, bb1 = biases(b_ih_1, b_hh_1)
    f1, r1 = _lstm_layer([f0, r0], wfs1, wbs1, combine_wh(wh1T),
                         bf1, bb1, bh, tc, False)

    wfs2, wbs2 = mk_io_weights(wi2T)
    bf2, bb2 = biases(b_ih_2, b_hh_2)
    (mean_out,) = _lstm_layer([f1, r1], wfs2, wbs2, combine_wh(wh2T),
                              bf2, bb2, bh, tc, True)
    return mean_out


# speedup vs baseline: 13.5502x; 1.0041x over previous
import functools

import jax
import jax.numpy as jnp
from jax.experimental import pallas as pl
from jax.experimental.pallas import tpu as pltpu

_NFILT = 40
_FPAD = 64


def _frontend_body(nfilt, b0_ref, b1_ref, b2_ref, x_ref, o_ref):
    nb = x_ref.shape[-1]
    fp = o_ref.shape[-1]
    b0, b1, b2 = b0_ref[...], b1_ref[...], b2_ref[...]
    f0, f1, f2 = jnp.floor(b0), jnp.floor(b1), jnp.floor(b2)
    i = jax.lax.broadcasted_iota(jnp.int32, (nb, fp), 0).astype(jnp.float32)
    j = jax.lax.broadcasted_iota(jnp.int32, (nb, fp), 1)
    rise_m = (i >= f0) & (i < f1)
    fall_m = (i >= f1) & (i < f2)
    d1 = b1 - b0
    d2 = b2 - b1
    rv = (i - b0) / jnp.where(d1 > 0, d1, 1.0) ** 2
    fv = (b2 - i) / jnp.where(d2 > 0, d2, 1.0) ** 2
    val = jnp.where(fall_m, fv, jnp.where(rise_m, rv, 0.0))
    val = jnp.where(j < nfilt - 1, val, 0.0)
    val = jnp.where(j == 0, jnp.where(i == 0.0, 1.0, 0.0), val)

    bh, tc, _ = x_ref.shape
    xb = x_ref[...].reshape(bh * tc, nb)
    filt = jnp.dot(xb, val, preferred_element_type=jnp.float32)
    h = jnp.log(filt + 1e-10).astype(jnp.bfloat16)
    o_ref[...] = h.reshape(bh, tc, fp)


def _frontend(x, binpoints, bh, tcf):
    B, T, NB = x.shape
    nt = T // tcf
    pad = _FPAD - _NFILT
    b0 = jnp.pad(binpoints[0:_NFILT], (0, pad)).reshape(1, _FPAD)
    b1 = jnp.pad(binpoints[1:_NFILT + 1], (0, pad)).reshape(1, _FPAD)
    b2 = jnp.pad(binpoints[2:_NFILT + 2], (0, pad)).reshape(1, _FPAD)
    return pl.pallas_call(
        functools.partial(_frontend_body, _NFILT),
        grid=(B // bh, nt),
        in_specs=[
            pl.BlockSpec((1, _FPAD), lambda b, k: (0, 0)),
            pl.BlockSpec((1, _FPAD), lambda b, k: (0, 0)),
            pl.BlockSpec((1, _FPAD), lambda b, k: (0, 0)),
            pl.BlockSpec((bh, tcf, NB), lambda b, k: (b, k, 0)),
        ],
        out_specs=pl.BlockSpec((bh, tcf, _FPAD), lambda b, k: (b, k, 0)),
        out_shape=jax.ShapeDtypeStruct((B, T, _FPAD), jnp.bfloat16),
        compiler_params=pltpu.CompilerParams(
            dimension_semantics=("parallel", "arbitrary")),
    )(b0, b1, b2, x)


def _lstm_body(tc, bh, hid2, n_in, accumulate, t_total, nt, *refs):
    g8 = 4 * hid2
    xf = refs[0:n_in]
    xb = refs[n_in:2 * n_in]
    wf = refs[2 * n_in:3 * n_in]
    wb = refs[3 * n_in:4 * n_in]
    wc, bf, bb = refs[4 * n_in:4 * n_in + 3]
    n_out = 1 if accumulate else 2
    outs = refs[4 * n_in + 3:4 * n_in + 3 + n_out]
    pf_s, pb_s, h_s, c_s, a_s = refs[4 * n_in + 3 + n_out:]
    k = pl.program_id(1)

    @pl.when(k == 0)
    def _():
        h_s[...] = jnp.zeros_like(h_s)
        c_s[...] = jnp.zeros_like(c_s)
        a_s[...] = jnp.zeros_like(a_s)

    def make_pre(xs, ws, b_ref):
        acc = b_ref[...]
        for xr, w_ref in zip(xs, ws):
            d = xr.shape[-1]
            x2 = xr[...].reshape(tc * bh, d)
            acc = acc + jnp.dot(x2, w_ref[...],
                                preferred_element_type=jnp.float32)
        return acc

    pf_s[...] = make_pre(xf, wf, bf)
    pb_s[...] = make_pre(xb, wb, bb)

    wc_v = wc[...]

    def step(t, carry):
        h, c, a = carry
        rf = pl.multiple_of(t * bh, bh)
        tb = tc - 1 - t
        rb = pl.multiple_of(tb * bh, bh)
        g = (pf_s[pl.ds(rf, bh), :] + pb_s[pl.ds(rb, bh), :]
             + jnp.dot(h, wc_v, preferred_element_type=jnp.float32))
        ig = jax.nn.sigmoid(g[:, 0:hid2])
        fg = jax.nn.sigmoid(g[:, hid2:2 * hid2])
        gg = jnp.tanh(g[:, 2 * hid2:3 * hid2])
        og = jax.nn.sigmoid(g[:, 3 * hid2:4 * hid2])
        c = fg * c + ig * gg
        hf32 = og * jnp.tanh(c)
        h = hf32.astype(jnp.bfloat16)
        if accumulate:
            a = a + hf32
        else:
            outs[0][pl.ds(t, 1)] = h.reshape(1, bh, hid2)
            outs[1][pl.ds(tb, 1)] = h.reshape(1, bh, hid2)
        return (h, c, a)

    init = (h_s[...], c_s[...], a_s[...])
    fin = jax.lax.fori_loop(0, tc, step, init)
    h_s[...], c_s[...], a_s[...] = fin

    if accumulate:
        @pl.when(k == nt - 1)
        def _():
            outs[0][...] = fin[2] * (1.0 / t_total)


def _lstm_layer(ins, wf_list, wb_list, wc, bf, bb, bh, tc, accumulate):
    T, B, _ = ins[0].shape
    hid2 = wc.shape[0]
    g8 = wc.shape[1]
    nt = T // tc
    nb = B // bh
    n_in = len(ins)

    in_specs = []
    args = []
    for xr in ins:
        d = xr.shape[-1]
        in_specs.append(pl.BlockSpec((tc, bh, d), lambda b, k: (k, b, 0)))
        args.append(xr)
    for xr in ins:
        d = xr.shape[-1]
        in_specs.append(
            pl.BlockSpec((tc, bh, d), lambda b, k: (nt - 1 - k, b, 0)))
        args.append(xr)
    for w in (*wf_list, *wb_list, wc, bf, bb):
        in_specs.append(pl.BlockSpec(w.shape, lambda b, k: (0,) * w.ndim))
        args.append(w)

    if accumulate:
        out_shape = (jax.ShapeDtypeStruct((B, hid2), jnp.float32),)
        out_specs = [pl.BlockSpec((bh, hid2), lambda b, k: (b, 0))]
    else:
        out_shape = (jax.ShapeDtypeStruct((T, B, hid2), jnp.bfloat16),) * 2
        out_specs = [
            pl.BlockSpec((tc, bh, hid2), lambda b, k: (k, b, 0)),
            pl.BlockSpec((tc, bh, hid2), lambda b, k: (nt - 1 - k, b, 0)),
        ]

    scratch = [
        pltpu.VMEM((tc * bh, g8), jnp.float32),
        pltpu.VMEM((tc * bh, g8), jnp.float32),
        pltpu.VMEM((bh, hid2), jnp.bfloat16),
        pltpu.VMEM((bh, hid2), jnp.float32),
        pltpu.VMEM((bh, hid2), jnp.float32),
    ]
    out = pl.pallas_call(
        functools.partial(_lstm_body, tc, bh, hid2, n_in, accumulate, T, nt),
        grid=(nb, nt),
        in_specs=in_specs,
        out_specs=out_specs,
        out_shape=out_shape,
        scratch_shapes=scratch,
        compiler_params=pltpu.CompilerParams(
            dimension_semantics=("parallel", "arbitrary"),
            vmem_limit_bytes=56 * 1024 * 1024),
    )(*args)
    return out


def _spread(w, slot, hid):
    z = jnp.zeros(w.shape[:-1] + (hid,), w.dtype)
    parts = []
    for q in range(4):
        blk = w[..., q * hid:(q + 1) * hid]
        parts.extend([blk, z] if slot == 0 else [z, blk])
    return jnp.concatenate(parts, axis=-1)


def kernel(x, binpoints, w_ih_0, w_hh_0, b_ih_0, b_hh_0,
           w_ih_1, w_hh_1, b_ih_1, b_hh_1,
           w_ih_2, w_hh_2, b_ih_2, b_hh_2):
    B, T, NB = x.shape
    hid = w_hh_0.shape[-1]
    bh = B // 2
    tcf = 200 if T % 200 == 0 else T
    tc = 100 if T % 100 == 0 else T

    h0 = _frontend(x, binpoints, bh, tcf)
    h0t = jnp.transpose(h0, (1, 0, 2))

    def wiT(w):
        return jnp.transpose(w, (0, 2, 1))

    wi0T = jnp.pad(wiT(w_ih_0), ((0, 0), (0, _FPAD - _NFILT), (0, 0)))
    wi1T, wi2T = wiT(w_ih_1), wiT(w_ih_2)
    wh0T, wh1T, wh2T = wiT(w_hh_0), wiT(w_hh_1), wiT(w_hh_2)

    def combine_wh(whT):
        return jnp.concatenate(
            [_spread(whT[0], 0, hid), _spread(whT[1], 1, hid)],
            axis=0).astype(jnp.bfloat16)

    def biases(b_ih, b_hh):
        bs = b_ih + b_hh
        return (_spread(bs[0].reshape(1, -1), 0, hid),
                _spread(bs[1].reshape(1, -1), 1, hid))

    def zero_rows(w, keep_top):
        top, bot = w[:hid], w[hid:]
        if keep_top:
            return jnp.concatenate([top, jnp.zeros_like(bot)], axis=0)
        return jnp.concatenate([jnp.zeros_like(top), bot], axis=0)

    bf0, bb0 = biases(b_ih_0, b_hh_0)
    f0, r0 = _lstm_layer(
        [h0t],
        [_spread(wi0T[0], 0, hid).astype(jnp.bfloat16)],
        [_spread(wi0T[1], 1, hid).astype(jnp.bfloat16)],
        combine_wh(wh0T), bf0, bb0, bh, tc, False)

    def mk_io_weights(wT):
        wfs = [_spread(zero_rows(wT[0], True), 0, hid).astype(jnp.bfloat16),
               _spread(zero_rows(wT[0], False), 0, hid).astype(jnp.bfloat16)]
        wbs = [_spread(zero_rows(wT[1], True), 1, hid).astype(jnp.bfloat16),
               _spread(zero_rows(wT[1], False), 1, hid).astype(jnp.bfloat16)]
        return wfs, wbs

    wfs1, wbs1 = mk_io_weights(wi1T)
    bf1, bb1 = biases(b_ih_1, b_hh_1)
    f1, r1 = _lstm_layer([f0, r0], wfs1, wbs1, combine_wh(wh1T),
                         bf1, bb1, bh, tc, False)

    wfs2, wbs2 = mk_io_weights(wi2T)
    bf2, bb2 = biases(b_ih_2, b_hh_2)
    (mean_out,) = _lstm_layer([f1, r1], wfs2, wbs2, combine_wh(wh2T),
                              bf2, bb2, bh, tc, True)
    return mean_out
